# baseline (speedup 1.0000x reference)
import functools

import jax
import jax.numpy as jnp
from jax import lax
from jax.experimental import pallas as pl
from jax.experimental.pallas import tpu as pltpu
from jax.experimental.pallas import tpu_sc as plsc

NC = 2
NS = 16
LANES = 16
NW = NC * NS

CH = 40
NP = 8


def _sc_gather_mean(input_ids, embed_table, B, S, D):
    BPW = B // NW
    DV = D // LANES
    NCH = S // CH

    mesh = plsc.VectorSubcoreMesh(core_axis_name="c", subcore_axis_name="s")

    @functools.partial(
        pl.kernel,
        out_type=jax.ShapeDtypeStruct((B, D), jnp.float32),
        mesh=mesh,
        scratch_types=[
            pltpu.VMEM((BPW * S,), jnp.int32),
            pltpu.VMEM((S, D), jnp.float32),
            pltpu.VMEM((S, D), jnp.float32),
            pltpu.VMEM((BPW, D), jnp.float32),
            pltpu.SemaphoreType.DMA,
            pltpu.SemaphoreType.DMA,
        ],
        compiler_params=pltpu.CompilerParams(use_tc_tiling_on_sc=False),
    )
    def sc_fn(idx_hbm, table_hbm, out_hbm, idx_v, buf_a, buf_b, out_v,
              sem_a, sem_b):
        wid = lax.axis_index("s") * NC + lax.axis_index("c")
        base = wid * BPW
        pltpu.sync_copy(idx_hbm.at[pl.ds(base * S, BPW * S)], idx_v)
        scale = jnp.float32(1.0 / S)

        def issue_row(r, buf, sem):
            for c in range(NCH):
                pltpu.async_copy(
                    table_hbm.at[idx_v.at[pl.ds(r * S + c * CH, CH)]],
                    buf.at[pl.ds(c * CH, CH)],
                    sem,
                )

        def drain_row(buf, sem):
            pltpu.make_async_copy(table_hbm.at[pl.ds(0, S)], buf, sem).wait()

        def compute_row(r, buf):
            for d in range(DV):
                p = [jnp.zeros((LANES,), jnp.float32) for _ in range(NP)]
                for j in range(S):
                    p[j % NP] = p[j % NP] + buf[j, pl.ds(d * LANES, LANES)]
                while len(p) > 1:
                    p = [p[i] + p[i + 1] for i in range(0, len(p), 2)]
                out_v[r, pl.ds(d * LANES, LANES)] = p[0] * scale

        issue_row(0, buf_a, sem_a)
        issue_row(1, buf_b, sem_b)

        def pair_fn(i, carry):
            r0 = 2 * i
            drain_row(buf_a, sem_a)
            compute_row(r0, buf_a)

            @pl.when(r0 + 2 < BPW)
            def _():
                issue_row(r0 + 2, buf_a, sem_a)

            drain_row(buf_b, sem_b)
            compute_row(r0 + 1, buf_b)

            @pl.when(r0 + 3 < BPW)
            def _():
                issue_row(r0 + 3, buf_b, sem_b)

            return carry

        lax.fori_loop(0, BPW // 2, pair_fn, 0)
        pltpu.sync_copy(out_v, out_hbm.at[pl.ds(base, BPW)])

    return sc_fn(input_ids, embed_table)


def _tc_linear_logsoftmax(x, W, b2, B, D, L):
    BT = 512

    def tc_body(x_ref, w_ref, b_ref, o_ref):
        logits = (
            jnp.dot(x_ref[...], w_ref[...], preferred_element_type=jnp.float32)
            + b_ref[...]
        )
        m = jnp.max(logits, axis=-1, keepdims=True)
        e = jnp.exp(logits - m)
        lse = jnp.log(jnp.sum(e, axis=-1, keepdims=True)) + m
        o_ref[...] = logits - lse

    return pl.pallas_call(
        tc_body,
        grid=(B // BT,),
        in_specs=[
            pl.BlockSpec((BT, D), lambda i: (i, 0)),
            pl.BlockSpec((D, L), lambda i: (0, 0)),
            pl.BlockSpec((1, L), lambda i: (0, 0)),
        ],
        out_specs=pl.BlockSpec((BT, L), lambda i: (i, 0)),
        out_shape=jax.ShapeDtypeStruct((B, L), jnp.float32),
    )(x, W, b2)


def kernel(input_ids, seq_len, embed_table, W, b):
    del seq_len
    B, S = input_ids.shape
    V, D = embed_table.shape
    L = W.shape[1]

    idx_flat = input_ids.astype(jnp.int32).reshape(B * S)
    pooled = _sc_gather_mean(idx_flat, embed_table, B, S, D)
    return _tc_linear_logsoftmax(pooled, W, b.reshape(1, L), B, D, L)


# speedup vs baseline: 1.0846x; 1.0846x over previous
import functools

import jax
import jax.numpy as jnp
from jax import lax
from jax.experimental import pallas as pl
from jax.experimental.pallas import tpu as pltpu
from jax.experimental.pallas import tpu_sc as plsc

NC = 2
NS = 16
LANES = 16
NW = NC * NS

CH = 40
NP = 8


def _sc_gather_mean(input_ids, embed_table, B, S, D):
    BPW = B // NW
    DV = D // LANES
    NCH = S // CH

    mesh = plsc.VectorSubcoreMesh(core_axis_name="c", subcore_axis_name="s")

    @functools.partial(
        pl.kernel,
        out_type=jax.ShapeDtypeStruct((B, D), jnp.float32),
        mesh=mesh,
        scratch_types=[
            pltpu.VMEM((BPW * S,), jnp.int32),
            pltpu.VMEM((S, D), jnp.float32),
            pltpu.VMEM((S, D), jnp.float32),
            pltpu.VMEM((BPW, D), jnp.float32),
            pltpu.SemaphoreType.DMA,
            pltpu.SemaphoreType.DMA,
        ],
        compiler_params=pltpu.CompilerParams(use_tc_tiling_on_sc=False),
    )
    def sc_fn(idx_hbm, table_hbm, out_hbm, idx_v, buf_a, buf_b, out_v,
              sem_a, sem_b):
        wid = lax.axis_index("s") * NC + lax.axis_index("c")
        base = wid * BPW
        pltpu.sync_copy(idx_hbm.at[pl.ds(base * S, BPW * S)], idx_v)
        scale = jnp.float32(1.0 / S)

        def issue_row(r, buf, sem):
            for c in range(NCH):
                pltpu.async_copy(
                    table_hbm.at[idx_v.at[pl.ds(r * S + c * CH, CH)]],
                    buf.at[pl.ds(c * CH, CH)],
                    sem,
                )

        def drain_row(buf, sem):
            pltpu.make_async_copy(table_hbm.at[pl.ds(0, S)], buf, sem).wait()

        def compute_row(r, buf):
            for d in range(DV):
                p = [jnp.zeros((LANES,), jnp.float32) for _ in range(NP)]
                for j in range(S):
                    p[j % NP] = p[j % NP] + buf[j, pl.ds(d * LANES, LANES)]
                while len(p) > 1:
                    p = [p[i] + p[i + 1] for i in range(0, len(p), 2)]
                out_v[r, pl.ds(d * LANES, LANES)] = p[0] * scale

        issue_row(0, buf_a, sem_a)
        issue_row(1, buf_b, sem_b)

        def pair_fn(i, carry):
            r0 = 2 * i
            drain_row(buf_a, sem_a)
            compute_row(r0, buf_a)

            @pl.when(r0 + 2 < BPW)
            def _():
                issue_row(r0 + 2, buf_a, sem_a)

            drain_row(buf_b, sem_b)
            compute_row(r0 + 1, buf_b)

            @pl.when(r0 + 3 < BPW)
            def _():
                issue_row(r0 + 3, buf_b, sem_b)

            return carry

        lax.fori_loop(0, BPW // 2, pair_fn, 0)
        pltpu.sync_copy(out_v, out_hbm.at[pl.ds(base, BPW)])

    return sc_fn(input_ids, embed_table)


def _tc_linear_logsoftmax(x, W, b2, B, D, L):
    BT = 512

    def tc_body(x_ref, w_ref, b_ref, o_ref):
        logits = (
            jnp.dot(x_ref[...], w_ref[...], preferred_element_type=jnp.float32)
            + b_ref[...]
        )
        m = jnp.max(logits, axis=-1, keepdims=True)
        e = jnp.exp(logits - m)
        lse = jnp.log(jnp.sum(e, axis=-1, keepdims=True)) + m
        o_ref[...] = logits - lse

    return pl.pallas_call(
        tc_body,
        grid=(B // BT,),
        in_specs=[
            pl.BlockSpec((BT, D), lambda i: (i, 0)),
            pl.BlockSpec((D, L), lambda i: (0, 0)),
            pl.BlockSpec((1, L), lambda i: (0, 0)),
        ],
        out_specs=pl.BlockSpec((BT, L), lambda i: (i, 0)),
        out_shape=jax.ShapeDtypeStruct((B, L), jnp.float32),
    )(x, W, b2)


def kernel(input_ids, seq_len, embed_table, W, b):
    del seq_len
    B, S = input_ids.shape
    V, D = embed_table.shape
    L = W.shape[1]

    table_pad = jnp.pad(embed_table, ((0, 0), (0, 64))).reshape(2 * V, D)
    idx_flat = input_ids.astype(jnp.int32).reshape(B * S) * 2
    pooled = _sc_gather_mean(idx_flat, table_pad, B, S, D)
    return _tc_linear_logsoftmax(pooled, W, b.reshape(1, L), B, D, L)


# speedup vs baseline: 1.1709x; 1.0796x over previous
import functools

import jax
import jax.numpy as jnp
from jax import lax
from jax.experimental import pallas as pl
from jax.experimental.pallas import tpu as pltpu
from jax.experimental.pallas import tpu_sc as plsc

NC = 2
NS = 16
LANES = 16
NW = NC * NS

CH = 40
NP = 8


def _sc_gather_mean(input_ids, embed_table, B, S, D):
    BPW = B // NW
    DV = D // LANES
    NCH = S // CH

    mesh = plsc.VectorSubcoreMesh(core_axis_name="c", subcore_axis_name="s")

    @functools.partial(
        pl.kernel,
        out_type=jax.ShapeDtypeStruct((B, D), jnp.float32),
        mesh=mesh,
        scratch_types=[
            pltpu.VMEM((BPW * S,), jnp.int32),
            pltpu.VMEM((S, D), jnp.float32),
            pltpu.VMEM((S, D), jnp.float32),
            pltpu.VMEM((BPW, D), jnp.float32),
            pltpu.SemaphoreType.DMA,
            pltpu.SemaphoreType.DMA,
        ],
        compiler_params=pltpu.CompilerParams(use_tc_tiling_on_sc=False),
    )
    def sc_fn(idx_hbm, table_hbm, out_hbm, idx_v, buf_a, buf_b, out_v,
              sem_a, sem_b):
        wid = lax.axis_index("s") * NC + lax.axis_index("c")
        base = wid * BPW
        pltpu.sync_copy(idx_hbm.at[pl.ds(base * S, BPW * S)], idx_v)
        scale = jnp.float32(1.0 / S)

        def issue_row(r, buf, sem):
            for c in range(NCH):
                pltpu.async_copy(
                    table_hbm.at[idx_v.at[pl.ds(r * S + c * CH, CH)]],
                    buf.at[pl.ds(c * CH, CH)],
                    sem,
                )

        def drain_row(buf, sem):
            pltpu.make_async_copy(table_hbm.at[pl.ds(0, S)], buf, sem).wait()

        def compute_row(r, buf):
            for d in range(DV):
                p = [jnp.zeros((LANES,), jnp.float32) for _ in range(NP)]
                for j in range(S):
                    p[j % NP] = p[j % NP] + buf[j, pl.ds(d * LANES, LANES)]
                while len(p) > 1:
                    p = [p[i] + p[i + 1] for i in range(0, len(p), 2)]
                out_v[r, pl.ds(d * LANES, LANES)] = p[0] * scale

        issue_row(0, buf_a, sem_a)
        issue_row(1, buf_b, sem_b)

        def pair_fn(i, carry):
            r0 = 2 * i
            drain_row(buf_a, sem_a)
            compute_row(r0, buf_a)

            @pl.when(r0 + 2 < BPW)
            def _():
                issue_row(r0 + 2, buf_a, sem_a)

            drain_row(buf_b, sem_b)
            compute_row(r0 + 1, buf_b)

            @pl.when(r0 + 3 < BPW)
            def _():
                issue_row(r0 + 3, buf_b, sem_b)

            return carry

        lax.fori_loop(0, BPW // 2, pair_fn, 0)
        pltpu.sync_copy(out_v, out_hbm.at[pl.ds(base, BPW)])

    return sc_fn(input_ids, embed_table)


def _tc_transpose_pad(tt, V, D):
    BT = 2048
    grid = (V + BT - 1) // BT

    def body(t_ref, o_ref):
        o_ref[:, :D] = t_ref[...].T
        o_ref[:, D:] = jnp.zeros((BT, 128 - D), jnp.float32)

    return pl.pallas_call(
        body,
        grid=(grid,),
        in_specs=[pl.BlockSpec((D, BT), lambda i: (0, i))],
        out_specs=pl.BlockSpec((BT, 128), lambda i: (i, 0)),
        out_shape=jax.ShapeDtypeStruct((V, 128), jnp.float32),
    )(tt)


def _tc_linear_logsoftmax(x, W, b2, B, D, L):
    BT = 512

    def tc_body(x_ref, w_ref, b_ref, o_ref):
        logits = (
            jnp.dot(x_ref[...], w_ref[...], preferred_element_type=jnp.float32)
            + b_ref[...]
        )
        m = jnp.max(logits, axis=-1, keepdims=True)
        e = jnp.exp(logits - m)
        lse = jnp.log(jnp.sum(e, axis=-1, keepdims=True)) + m
        o_ref[...] = logits - lse

    return pl.pallas_call(
        tc_body,
        grid=(B // BT,),
        in_specs=[
            pl.BlockSpec((BT, D), lambda i: (i, 0)),
            pl.BlockSpec((D, L), lambda i: (0, 0)),
            pl.BlockSpec((1, L), lambda i: (0, 0)),
        ],
        out_specs=pl.BlockSpec((BT, L), lambda i: (i, 0)),
        out_shape=jax.ShapeDtypeStruct((B, L), jnp.float32),
    )(x, W, b2)


def kernel(input_ids, seq_len, embed_table, W, b):
    del seq_len
    B, S = input_ids.shape
    V, D = embed_table.shape
    L = W.shape[1]

    table_pad = _tc_transpose_pad(embed_table.T, V, D).reshape(2 * V, D)
    idx_flat = input_ids.astype(jnp.int32).reshape(B * S) * 2
    pooled = _sc_gather_mean(idx_flat, table_pad, B, S, D)
    return _tc_linear_logsoftmax(pooled, W, b.reshape(1, L), B, D, L)
